# baseline (speedup 1.0000x reference)
import functools

import jax
import jax.numpy as jnp
from jax import lax
from jax.experimental import pallas as pl
from jax.experimental.pallas import tpu as pltpu
from jax.experimental.pallas import tpu_sc as plsc

_B, _ZD, _H, _W = 32, 64, 32, 32
_M = 1024
_MU = 0.99
_N = _B * _H * _W
_BN = 1024
_NBLK = _N // _BN
_NSUB = 16
_PPS = _N // _NSUB
_CHUNK = 512
_MROWS = _M // _NSUB


def _bdot(a, b, dims=(((1,), (0,)), ((), ()))):
    return jax.lax.dot_general(a, b, dims,
                               preferred_element_type=jnp.float32)


def _prelude(psum_ref, pelem_col_ref, w4_ref, b4_ref,
             e_out, g_out, c_out):
    e = psum_ref[...] / pelem_col_ref[...]
    e_out[...] = e
    g_out[...] = (-2.0 * _bdot(w4_ref[...].astype(jnp.float32), e,
                               (((1,), (1,)), ((), ())))).astype(jnp.bfloat16)
    c_out[...] = jnp.sum(e * (e - 2.0 * b4_ref[...]), axis=1)[None, :]


def _body(x_ref, pelem_row_ref,
          w1_ref, b1_ref, w2_ref, b2_ref, w3_ref, b3_ref, w4_ref, b4_ref,
          g_ref, c_ref, msplit_ref,
          z_out, belong_out, zi_out, pe_out):
    i = pl.program_id(0)

    @pl.when(i == 0)
    def _init():
        pe_out[...] = _MU * pelem_row_ref[...]

    x = x_ref[...].astype(jnp.bfloat16)
    h = jnp.tanh(_bdot(x, w1_ref[...]) + b1_ref[...]).astype(jnp.bfloat16)
    h = jnp.tanh(_bdot(h, w2_ref[...]) + b2_ref[...]).astype(jnp.bfloat16)
    h = jnp.tanh(_bdot(h, w3_ref[...]) + b3_ref[...]).astype(jnp.bfloat16)
    zz = _bdot(h, w4_ref[...]) + b4_ref[...]
    z_out[...] = zz

    dist = _bdot(h, g_ref[...]) + c_ref[...]

    dmin = jnp.min(dist, axis=1, keepdims=True)
    onehot = jnp.where(dist <= dmin, 1.0, 0.0)
    belong_out[...] = onehot

    s2 = _bdot(onehot.astype(jnp.bfloat16), msplit_ref[...])
    zi_out[...] = (32.0 * s2[:, 0:1] + s2[:, 1:2]).astype(jnp.int32)
    pe_out[...] += (1.0 - _MU) * jnp.sum(onehot, axis=0, keepdims=True)


def _sc_update(zi_ref, zz_ref, psum_ref, zps_ref,
               psn_ref,
               idx_v, zz_v, accv, priv, outv,
               acc_ps):
    c = lax.axis_index("c")
    s = lax.axis_index("s")

    @pl.when(jnp.logical_and(c == 0, s == 0))
    def _():
        pltpu.sync_copy(zps_ref, acc_ps)

    plsc.subcore_barrier()

    @pl.when(c == 0)
    def _():
        for k in range(_PPS // _CHUNK):
            base = pl.multiple_of(s * _PPS + k * _CHUNK, _CHUNK)
            pltpu.sync_copy(zz_ref.at[pl.ds(base, _CHUNK)], zz_v)
            for j in range(_CHUNK // 128):
                pltpu.sync_copy(zi_ref.at[pl.ds(base + j * 128, 128)], idx_v)
                pltpu.sync_copy(zz_v.at[pl.ds(j * 128, 128)],
                                acc_ps.at[idx_v], add=True)

    plsc.subcore_barrier()

    @pl.when(c == 0)
    def _():
        r0 = pl.multiple_of(s * _MROWS, _MROWS)
        pltpu.sync_copy(acc_ps.at[pl.ds(r0, _MROWS)], accv)
        pltpu.sync_copy(psum_ref.at[pl.ds(r0, _MROWS)], priv)

        def row(r, carry):
            for cc in range(_ZD // 16):
                sl = (r, pl.ds(cc * 16, 16))
                outv[sl] = _MU * priv[sl] + (1.0 - _MU) * accv[sl]
            return carry
        lax.fori_loop(0, _MROWS, row, 0)

        pltpu.sync_copy(outv, psn_ref.at[pl.ds(r0, _MROWS)])


@functools.partial(jax.jit, static_argnames=("interpret",))
def kernel(z, prior_sum, prior_elem, W1, b1, W2, b2, W3, b3, W4, b4,
           interpret=False):
    x = jnp.transpose(z, (0, 2, 3, 1)).reshape(_N, _ZD)
    pelem_col = prior_elem.reshape(_M, 1)
    bf = jnp.bfloat16
    w1, w2, w3, w4 = W1.T.astype(bf), W2.T.astype(bf), W3.T.astype(bf), W4.T.astype(bf)
    mm = jnp.arange(_M, dtype=jnp.int32)
    msplit = jnp.stack([mm // 32, mm % 32], axis=1).astype(bf)

    full = lambda shape: pl.BlockSpec(shape, lambda *_: tuple(0 for _ in shape))

    e, g, c = pl.pallas_call(
        _prelude,
        in_specs=[full((_M, _ZD)), full((_M, 1)), full((_ZD * 4, _ZD)),
                  full((1, _ZD))],
        out_specs=[full((_M, _ZD)), full((_ZD * 4, _M)), full((1, _M))],
        out_shape=[jax.ShapeDtypeStruct((_M, _ZD), jnp.float32),
                   jax.ShapeDtypeStruct((_ZD * 4, _M), jnp.bfloat16),
                   jax.ShapeDtypeStruct((1, _M), jnp.float32)],
        interpret=interpret,
    )(prior_sum, pelem_col, w4, b4.reshape(1, -1))

    zflat, belong, zi, pe_new = pl.pallas_call(
        _body,
        grid=(_NBLK,),
        in_specs=[
            pl.BlockSpec((_BN, _ZD), lambda i: (i, 0)),
            full((1, _M)),
            full((_ZD, _ZD * 4)), full((1, _ZD * 4)),
            full((_ZD * 4, _ZD * 4)), full((1, _ZD * 4)),
            full((_ZD * 4, _ZD * 4)), full((1, _ZD * 4)),
            full((_ZD * 4, _ZD)), full((1, _ZD)),
            full((_ZD * 4, _M)),
            full((1, _M)),
            full((_M, 2)),
        ],
        out_specs=[
            pl.BlockSpec((_BN, _ZD), lambda i: (i, 0)),
            pl.BlockSpec((_BN, _M), lambda i: (i, 0)),
            pl.BlockSpec((_BN, 1), lambda i: (i, 0)),
            full((1, _M)),
        ],
        out_shape=[jax.ShapeDtypeStruct((_N, _ZD), jnp.float32),
                   jax.ShapeDtypeStruct((_N, _M), jnp.float32),
                   jax.ShapeDtypeStruct((_N, 1), jnp.int32),
                   jax.ShapeDtypeStruct((1, _M), jnp.float32)],
        interpret=interpret,
    )(x, prior_elem.reshape(1, _M),
      w1, b1.reshape(1, -1), w2, b2.reshape(1, -1),
      w3, b3.reshape(1, -1), w4, b4.reshape(1, -1),
      g, c, msplit)

    zi1d = zi.reshape(_N)
    sc = pl.kernel(
        _sc_update,
        out_type=jax.ShapeDtypeStruct((_M, _ZD), jnp.float32),
        mesh=plsc.VectorSubcoreMesh(core_axis_name="c", subcore_axis_name="s"),
        scratch_types=[
            pltpu.VMEM((128,), jnp.int32),
            pltpu.VMEM((_CHUNK, _ZD), jnp.float32),
            pltpu.VMEM((_MROWS, _ZD), jnp.float32),
            pltpu.VMEM((_MROWS, _ZD), jnp.float32),
            pltpu.VMEM((_MROWS, _ZD), jnp.float32),
            pltpu.VMEM_SHARED((_M, _ZD), jnp.float32),
        ],
    )
    ps_new = sc(zi1d, zflat, prior_sum,
                jnp.zeros((_M, _ZD), jnp.float32))

    z_out = jnp.transpose(zflat.reshape(_B, _H, _W, _ZD), (0, 3, 1, 2))
    return (e, z_out, belong, ps_new, pe_new.reshape(_M))


# speedup vs baseline: 1.6303x; 1.6303x over previous
import functools

import jax
import jax.numpy as jnp
from jax.experimental import pallas as pl
from jax.experimental.pallas import tpu as pltpu

_B, _ZD, _H, _W = 32, 64, 32, 32
_M = 1024
_MU = 0.99
_N = _B * _H * _W
_BN = 2048
_NBLK = _N // _BN


def _bdot(a, b, dims=(((1,), (0,)), ((), ()))):
    return jax.lax.dot_general(a, b, dims,
                               preferred_element_type=jnp.float32)


def _prelude(psum_ref, pelem_col_ref, w4_ref, b4_ref,
             e_out, g_out, c_out):
    e = psum_ref[...] / pelem_col_ref[...]
    e_out[...] = e
    g_out[...] = (-2.0 * _bdot(w4_ref[...].astype(jnp.float32), e,
                               (((1,), (1,)), ((), ())))).astype(jnp.bfloat16)
    c_out[...] = jnp.sum(e * (e - 2.0 * b4_ref[...]), axis=1)[None, :]


def _body(x_ref, psum_ref, pelem_row_ref,
          w1_ref, b1_ref, w2_ref, b2_ref, w3_ref, b3_ref, w4_ref, b4_ref,
          g_ref, c_ref,
          z_out, belong_out, ps_out, pe_out):
    i = pl.program_id(0)

    @pl.when(i == 0)
    def _init():
        ps_out[...] = _MU * psum_ref[...]
        pe_out[...] = _MU * pelem_row_ref[...]

    x = x_ref[...].astype(jnp.bfloat16)
    h = jnp.tanh(_bdot(x, w1_ref[...]) + b1_ref[...]).astype(jnp.bfloat16)
    h = jnp.tanh(_bdot(h, w2_ref[...]) + b2_ref[...]).astype(jnp.bfloat16)
    h = jnp.tanh(_bdot(h, w3_ref[...]) + b3_ref[...]).astype(jnp.bfloat16)
    zz = _bdot(h, w4_ref[...]) + b4_ref[...]
    z_out[...] = zz

    dist = _bdot(h, g_ref[...]) + c_ref[...]

    dmin = jnp.min(dist, axis=1, keepdims=True)
    onehot = jnp.where(dist <= dmin, 1.0, 0.0)
    belong_out[...] = onehot

    ps_out[...] += (1.0 - _MU) * _bdot(
        onehot.astype(jnp.bfloat16), zz.astype(jnp.bfloat16),
        (((0,), (0,)), ((), ())))
    pe_out[...] += (1.0 - _MU) * jnp.sum(onehot, axis=0, keepdims=True)


@functools.partial(jax.jit, static_argnames=("interpret",))
def kernel(z, prior_sum, prior_elem, W1, b1, W2, b2, W3, b3, W4, b4,
           interpret=False):
    x = jnp.transpose(z, (0, 2, 3, 1)).reshape(_N, _ZD)
    pelem_col = prior_elem.reshape(_M, 1)
    pelem_row = prior_elem.reshape(1, _M)
    bf = jnp.bfloat16
    w1, w2, w3, w4 = W1.T.astype(bf), W2.T.astype(bf), W3.T.astype(bf), W4.T.astype(bf)

    full = lambda shape: pl.BlockSpec(shape, lambda *_: tuple(0 for _ in shape))

    e, g, c = pl.pallas_call(
        _prelude,
        in_specs=[full((_M, _ZD)), full((_M, 1)), full((_ZD * 4, _ZD)),
                  full((1, _ZD))],
        out_specs=[full((_M, _ZD)), full((_ZD * 4, _M)), full((1, _M))],
        out_shape=[jax.ShapeDtypeStruct((_M, _ZD), jnp.float32),
                   jax.ShapeDtypeStruct((_ZD * 4, _M), jnp.bfloat16),
                   jax.ShapeDtypeStruct((1, _M), jnp.float32)],
        interpret=interpret,
    )(prior_sum, pelem_col, w4, b4.reshape(1, -1))

    zflat, belong, ps_new, pe_new = pl.pallas_call(
        _body,
        grid=(_NBLK,),
        in_specs=[
            pl.BlockSpec((_BN, _ZD), lambda i: (i, 0)),
            full((_M, _ZD)),
            full((1, _M)),
            full((_ZD, _ZD * 4)), full((1, _ZD * 4)),
            full((_ZD * 4, _ZD * 4)), full((1, _ZD * 4)),
            full((_ZD * 4, _ZD * 4)), full((1, _ZD * 4)),
            full((_ZD * 4, _ZD)), full((1, _ZD)),
            full((_ZD * 4, _M)),
            full((1, _M)),
        ],
        out_specs=[
            pl.BlockSpec((_BN, _ZD), lambda i: (i, 0)),
            pl.BlockSpec((_BN, _M), lambda i: (i, 0)),
            full((_M, _ZD)),
            full((1, _M)),
        ],
        out_shape=[jax.ShapeDtypeStruct((_N, _ZD), jnp.float32),
                   jax.ShapeDtypeStruct((_N, _M), jnp.float32),
                   jax.ShapeDtypeStruct((_M, _ZD), jnp.float32),
                   jax.ShapeDtypeStruct((1, _M), jnp.float32)],
        interpret=interpret,
    )(x, prior_sum, pelem_row,
      w1, b1.reshape(1, -1), w2, b2.reshape(1, -1),
      w3, b3.reshape(1, -1), w4, b4.reshape(1, -1),
      g, c)

    z_out = jnp.transpose(zflat.reshape(_B, _H, _W, _ZD), (0, 3, 1, 2))
    return (e, z_out, belong, ps_new, pe_new.reshape(_M))
